# baseline (speedup 1.0000x reference)
import functools

import jax
import jax.numpy as jnp
from jax import lax
from jax.experimental import pallas as pl
from jax.experimental.pallas import tpu as pltpu
from jax.experimental.pallas import tpu_sc as plsc

NUM_L0 = 4
NUM_L1 = 16
NUM_L2 = 16
TOKENS_PER_TIMESTEP = NUM_L0 + NUM_L1 + NUM_L2 + 1
B = 1024
T = 16
D = 384
SEQ = T * TOKENS_PER_TIMESTEP
ROWS_TOTAL = B * SEQ

TSTRIDE = 2128
OFF_L1 = 64
OFF_L2 = 1088
OFF_ACT = 2112

NC, NS = 2, 16
NW = NC * NS
ROWS_PER_W = ROWS_TOTAL // NW
CHUNK = 64
NCHUNK = ROWS_PER_W // CHUNK
NBUF = 2
NOUTER = NCHUNK // NBUF


def _build_body(l0, l1, l2, act, lvl, patch, pos_r, out):
    base_l0 = l0[...] + lvl[0:1, :]
    for j in range(NUM_L0):
        out[0, j * 16:(j + 1) * 16, :] = (
            base_l0 + (patch[j:j + 1, :] + pos_r[0, j:j + 1, :]))
    base_l1 = l1[...] + lvl[1:2, :]
    for j in range(NUM_L1):
        out[0, OFF_L1 + j * 64:OFF_L1 + (j + 1) * 64, :] = (
            base_l1 + (patch[j:j + 1, :] + pos_r[0, 4 + j:5 + j, :]))
    base_l2 = l2[...] + lvl[2:3, :]
    for j in range(NUM_L2):
        out[0, OFF_L2 + j * 64:OFF_L2 + (j + 1) * 64, :] = (
            base_l2 + (patch[j:j + 1, :] + pos_r[0, 20 + j:21 + j, :]))
    act_block = jnp.concatenate(
        [act[...] + (lvl[3:4, :] + pos_r[0, 36:37, :]),
         jnp.zeros((16 - 9, D), jnp.float32)], axis=0)
    out[0, OFF_ACT:OFF_ACT + 16, :] = act_block


_build_ftab = pl.pallas_call(
    _build_body,
    grid=(T,),
    in_specs=[
        pl.BlockSpec((16, D), lambda t: (0, 0)),
        pl.BlockSpec((64, D), lambda t: (0, 0)),
        pl.BlockSpec((64, D), lambda t: (0, 0)),
        pl.BlockSpec((9, D), lambda t: (0, 0)),
        pl.BlockSpec((4, D), lambda t: (0, 0)),
        pl.BlockSpec((16, D), lambda t: (0, 0)),
        pl.BlockSpec((1, TOKENS_PER_TIMESTEP, D), lambda t: (t, 0, 0)),
    ],
    out_specs=pl.BlockSpec((1, TSTRIDE, D), lambda t: (t, 0, 0)),
    out_shape=jax.ShapeDtypeStruct((T, TSTRIDE, D), jnp.float32),
)

@functools.cache
def _make_sc_gather():
    mesh = plsc.VectorSubcoreMesh(core_axis_name="c", subcore_axis_name="s",
                                  num_cores=NC, num_subcores=NS)
    return functools.partial(
        pl.kernel,
        out_type=jax.ShapeDtypeStruct((ROWS_TOTAL, D), jnp.float32),
        mesh=mesh,
        scratch_types=[
        pltpu.VMEM((NCHUNK, CHUNK), jnp.int32),
        pltpu.VMEM((CHUNK, D), jnp.float32),
        pltpu.VMEM((CHUNK, D), jnp.float32),
        pltpu.SemaphoreType.DMA,
        pltpu.SemaphoreType.DMA,
        pltpu.SemaphoreType.DMA,
        pltpu.SemaphoreType.DMA,
    ],
    )(_sc_gather_body)


def _sc_gather_body(ftab_hbm, gidx_hbm, out_hbm, idx_v,
               buf0, buf1, g0, g1, s0, s1):
    bufs = (buf0, buf1)
    gsems = (g0, g1)
    ssems = (s0, s1)
    wid = lax.axis_index("s") * NC + lax.axis_index("c")
    base = wid * ROWS_PER_W
    pltpu.sync_copy(gidx_hbm.at[wid], idx_v)

    def g_desc(c, b):
        return pltpu.make_async_copy(ftab_hbm.at[idx_v.at[c]], bufs[b],
                                     gsems[b])

    def s_desc(c, b):
        return pltpu.make_async_copy(
            bufs[b], out_hbm.at[pl.ds(base + c * CHUNK, CHUNK)], ssems[b])

    for b in range(NBUF):
        g_desc(b, b).start()

    def body(i, carry):
        for b in range(NBUF):
            c = i * NBUF + b
            g_desc(c, b).wait()
            s_desc(c, b).start()
            pb = (b - 1) % NBUF
            cond = (i > 0) if b == 0 else (i < NOUTER - 1)

            @pl.when(cond)
            def _(c=c, pb=pb):
                s_desc(c - 1, pb).wait()
                g_desc(c - 1 + NBUF, pb).start()

        return carry

    lax.fori_loop(0, NOUTER, body, 0)
    for b in range(NBUF):
        s_desc(NCHUNK - NBUF + b, b).wait()


def kernel(tokens_l0, tokens_l1, tokens_l2, actions, l0_embed, l1_embed,
           l2_embed, act_embed, level_embed, patch_embed, pos_embed):
    ftab = _build_ftab(l0_embed, l1_embed, l2_embed, act_embed, level_embed,
                       patch_embed, pos_embed.reshape(T, TOKENS_PER_TIMESTEP, D))
    ftab_flat = ftab.reshape(T * TSTRIDE, D)

    g0 = tokens_l0 + jnp.arange(NUM_L0, dtype=jnp.int32) * 16
    g1 = tokens_l1 + (OFF_L1 + jnp.arange(NUM_L1, dtype=jnp.int32) * 64)
    g2 = tokens_l2 + (OFF_L2 + jnp.arange(NUM_L2, dtype=jnp.int32) * 64)
    ga = actions[:, :, None] + OFF_ACT
    gidx = jnp.concatenate([g0, g1, g2, ga], axis=2)
    gidx = gidx + (jnp.arange(T, dtype=jnp.int32) * TSTRIDE)[None, :, None]
    gidx = gidx.reshape(NW, NCHUNK, CHUNK)

    out = _make_sc_gather()(ftab_flat, gidx)
    return out.reshape(B, SEQ, D)


# speedup vs baseline: 1.0552x; 1.0552x over previous
import functools

import jax
import jax.numpy as jnp
from jax import lax
from jax.experimental import pallas as pl
from jax.experimental.pallas import tpu as pltpu
from jax.experimental.pallas import tpu_sc as plsc

NUM_L0 = 4
NUM_L1 = 16
NUM_L2 = 16
TOKENS_PER_TIMESTEP = NUM_L0 + NUM_L1 + NUM_L2 + 1
B = 1024
T = 16
D = 384
SEQ = T * TOKENS_PER_TIMESTEP
ROWS_TOTAL = B * SEQ

TSTRIDE = 2128
OFF_L1 = 64
OFF_L2 = 1088
OFF_ACT = 2112

NC, NS = 2, 16
NW = NC * NS
ROWS_PER_W = ROWS_TOTAL // NW
CHUNK = 32
NCHUNK = ROWS_PER_W // CHUNK
CPR = 128 // CHUNK
NBUF = 8
NOUTER = NCHUNK // NBUF


def _build_body(l0, l1, l2, act, lvl, patch, pos_r, out):
    base_l0 = l0[...] + lvl[0:1, :]
    for j in range(NUM_L0):
        out[0, j * 16:(j + 1) * 16, :] = (
            base_l0 + (patch[j:j + 1, :] + pos_r[0, j:j + 1, :]))
    base_l1 = l1[...] + lvl[1:2, :]
    for j in range(NUM_L1):
        out[0, OFF_L1 + j * 64:OFF_L1 + (j + 1) * 64, :] = (
            base_l1 + (patch[j:j + 1, :] + pos_r[0, 4 + j:5 + j, :]))
    base_l2 = l2[...] + lvl[2:3, :]
    for j in range(NUM_L2):
        out[0, OFF_L2 + j * 64:OFF_L2 + (j + 1) * 64, :] = (
            base_l2 + (patch[j:j + 1, :] + pos_r[0, 20 + j:21 + j, :]))
    act_block = jnp.concatenate(
        [act[...] + (lvl[3:4, :] + pos_r[0, 36:37, :]),
         jnp.zeros((16 - 9, D), jnp.float32)], axis=0)
    out[0, OFF_ACT:OFF_ACT + 16, :] = act_block


_build_ftab = pl.pallas_call(
    _build_body,
    grid=(T,),
    in_specs=[
        pl.BlockSpec((16, D), lambda t: (0, 0)),
        pl.BlockSpec((64, D), lambda t: (0, 0)),
        pl.BlockSpec((64, D), lambda t: (0, 0)),
        pl.BlockSpec((9, D), lambda t: (0, 0)),
        pl.BlockSpec((4, D), lambda t: (0, 0)),
        pl.BlockSpec((16, D), lambda t: (0, 0)),
        pl.BlockSpec((1, TOKENS_PER_TIMESTEP, D), lambda t: (t, 0, 0)),
    ],
    out_specs=pl.BlockSpec((1, TSTRIDE, D), lambda t: (t, 0, 0)),
    out_shape=jax.ShapeDtypeStruct((T, TSTRIDE, D), jnp.float32),
)

@functools.cache
def _make_sc_gather():
    mesh = plsc.VectorSubcoreMesh(core_axis_name="c", subcore_axis_name="s",
                                  num_cores=NC, num_subcores=NS)
    return functools.partial(
        pl.kernel,
        out_type=jax.ShapeDtypeStruct((ROWS_TOTAL, D), jnp.float32),
        mesh=mesh,
        scratch_types=[
        pltpu.VMEM((NCHUNK // CPR, 128), jnp.int32),
    ] + [pltpu.VMEM((CHUNK, D), jnp.float32)] * NBUF
      + [pltpu.SemaphoreType.DMA] * (2 * NBUF),
    )(_sc_gather_body)


def _sc_gather_body(ftab_hbm, gidx_hbm, out_hbm, idx_v, *bufs_sems):
    bufs = bufs_sems[:NBUF]
    gsems = bufs_sems[NBUF:2 * NBUF]
    ssems = bufs_sems[2 * NBUF:]
    wid = lax.axis_index("s") * NC + lax.axis_index("c")
    base = wid * ROWS_PER_W
    pltpu.sync_copy(gidx_hbm.at[wid], idx_v)

    def g_desc(c, b):
        idx_slice = idx_v.at[c // CPR, pl.ds((c % CPR) * CHUNK, CHUNK)]
        return pltpu.make_async_copy(ftab_hbm.at[idx_slice], bufs[b],
                                     gsems[b])

    def s_desc(c, b):
        return pltpu.make_async_copy(
            bufs[b], out_hbm.at[pl.ds(base + c * CHUNK, CHUNK)], ssems[b])

    for b in range(NBUF):
        g_desc(b, b).start()

    def body(i, carry):
        for b in range(NBUF):
            c = i * NBUF + b
            g_desc(c, b).wait()
            s_desc(c, b).start()
            pb = (b - 1) % NBUF
            cond = (i > 0) if b == 0 else (i < NOUTER - 1)

            @pl.when(cond)
            def _(c=c, pb=pb):
                s_desc(c - 1, pb).wait()
                g_desc(c - 1 + NBUF, pb).start()

        return carry

    lax.fori_loop(0, NOUTER, body, 0)
    for b in range(NBUF):
        s_desc(NCHUNK - NBUF + b, b).wait()


def kernel(tokens_l0, tokens_l1, tokens_l2, actions, l0_embed, l1_embed,
           l2_embed, act_embed, level_embed, patch_embed, pos_embed):
    ftab = _build_ftab(l0_embed, l1_embed, l2_embed, act_embed, level_embed,
                       patch_embed, pos_embed.reshape(T, TOKENS_PER_TIMESTEP, D))
    ftab_flat = ftab.reshape(T * TSTRIDE, D)

    g0 = tokens_l0 + jnp.arange(NUM_L0, dtype=jnp.int32) * 16
    g1 = tokens_l1 + (OFF_L1 + jnp.arange(NUM_L1, dtype=jnp.int32) * 64)
    g2 = tokens_l2 + (OFF_L2 + jnp.arange(NUM_L2, dtype=jnp.int32) * 64)
    ga = actions[:, :, None] + OFF_ACT
    gidx = jnp.concatenate([g0, g1, g2, ga], axis=2)
    gidx = gidx + (jnp.arange(T, dtype=jnp.int32) * TSTRIDE)[None, :, None]
    gidx = gidx.reshape(NW, NCHUNK // CPR, 128)

    out = _make_sc_gather()(ftab_flat, gidx)
    return out.reshape(B, SEQ, D)
